# baseline (speedup 1.0000x reference)
import functools

import jax
import jax.numpy as jnp
from jax import lax
from jax.experimental import pallas as pl
from jax.experimental.pallas import tpu as pltpu
from jax.experimental.pallas import tpu_sc as plsc

N_NODE = 1000000
EMBED_DIM = 16
BATCH = 16384

NUM_CORES = 2
NUM_SUBCORES = 16
NUM_LANES = 16
NW = NUM_CORES * NUM_SUBCORES
B_PER_W = BATCH // NW
NBLK = B_PER_W // NUM_LANES


def _sc_body(table_t, bias_tab, nid, nnid,
             score_out, embu_t_out, embv_t_out, bias_out,
             idx_u, idx_v, cols_u, cols_v, bias_vm, score_vm,
             sem_u, sem_v, sem_b, sem_o):
    wid = lax.axis_index("s") * NUM_CORES + lax.axis_index("c")
    base = wid * B_PER_W

    pltpu.sync_copy(nid.at[pl.ds(base, B_PER_W)], idx_u)
    pltpu.sync_copy(nnid.at[pl.ds(base, B_PER_W)], idx_v)
    cp_b = pltpu.async_copy(bias_tab.at[idx_v], bias_vm, sem_b)
    cps = []
    for d in range(EMBED_DIM):
        cps.append(pltpu.async_copy(table_t.at[d].at[idx_u],
                                    cols_u.at[d], sem_u))
        cps.append(pltpu.async_copy(table_t.at[d].at[idx_v],
                                    cols_v.at[d], sem_v))
    for cp in cps:
        cp.wait()
    cp_b.wait()

    ocs = [pltpu.async_copy(bias_vm, bias_out.at[pl.ds(base, B_PER_W)], sem_o)]
    for d in range(EMBED_DIM):
        ocs.append(pltpu.async_copy(
            cols_u.at[d], embu_t_out.at[d, pl.ds(base, B_PER_W)], sem_o))
        ocs.append(pltpu.async_copy(
            cols_v.at[d], embv_t_out.at[d, pl.ds(base, B_PER_W)], sem_o))

    def block(j, carry):
        sl = pl.ds(j * NUM_LANES, NUM_LANES)
        acc = bias_vm[sl]
        for d in range(EMBED_DIM):
            acc = acc + cols_u[d, sl] * cols_v[d, sl]
        score_vm[sl] = jnp.clip(acc, -10.0, 10.0)
        return carry

    lax.fori_loop(0, NBLK, block, 0)

    pltpu.sync_copy(score_vm, score_out.at[pl.ds(base, B_PER_W)])
    for oc in ocs:
        oc.wait()


@jax.jit
def kernel(embedding_matrix, bias_vector, node_id, node_neighbor_id):
    mesh = plsc.VectorSubcoreMesh(core_axis_name="c", subcore_axis_name="s")
    f = functools.partial(
        pl.kernel,
        mesh=mesh,
        compiler_params=pltpu.CompilerParams(use_tc_tiling_on_sc=False),
        out_type=[
            jax.ShapeDtypeStruct((BATCH,), jnp.float32),
            jax.ShapeDtypeStruct((EMBED_DIM, BATCH), jnp.float32),
            jax.ShapeDtypeStruct((EMBED_DIM, BATCH), jnp.float32),
            jax.ShapeDtypeStruct((BATCH,), jnp.float32),
        ],
        scratch_types=[
            pltpu.VMEM((B_PER_W,), jnp.int32),
            pltpu.VMEM((B_PER_W,), jnp.int32),
            pltpu.VMEM((EMBED_DIM, B_PER_W), jnp.float32),
            pltpu.VMEM((EMBED_DIM, B_PER_W), jnp.float32),
            pltpu.VMEM((B_PER_W,), jnp.float32),
            pltpu.VMEM((B_PER_W,), jnp.float32),
            pltpu.SemaphoreType.DMA,
            pltpu.SemaphoreType.DMA,
            pltpu.SemaphoreType.DMA,
            pltpu.SemaphoreType.DMA,
        ],
    )(_sc_body)
    score, embu_t, embv_t, bias = f(
        embedding_matrix.T,
        bias_vector,
        node_id.astype(jnp.int32),
        node_neighbor_id.astype(jnp.int32),
    )
    return (score, embu_t.T, embv_t.T, bias)


# speedup vs baseline: 12.9819x; 12.9819x over previous
import functools

import jax
import jax.numpy as jnp
from jax import lax
from jax.experimental import pallas as pl
from jax.experimental.pallas import tpu as pltpu
from jax.experimental.pallas import tpu_sc as plsc

N_NODE = 1000000
EMBED_DIM = 16
BATCH = 16384

NUM_CORES = 2
NUM_SUBCORES = 16
NUM_LANES = 16
NW = NUM_CORES * NUM_SUBCORES
B_PER_W = BATCH // NW
NBLK = B_PER_W // NUM_LANES

GRP = 8
WIN = 12800
NFULL = N_NODE // WIN
WIN2 = 1536
COVER = NFULL * WIN + WIN2
TAIL = N_NODE - COVER
STRIDE = 7813 * 128
NITEM = (NFULL + 1) * 2 + 1
ITER = -(-NITEM // NW)


def _detile_body(table_t, tail_t, flat_out, buf, buf2, tail_buf):
    wid = lax.axis_index("s") * NUM_CORES + lax.axis_index("c")

    def round_(k, carry):
        item = wid + k * NW

        @pl.when(item < NITEM - 1)
        def _():
            g = item % 2
            win = item // 2
            r0 = g * GRP

            @pl.when(win < NFULL)
            def _full():
                c0 = pl.multiple_of(win * WIN, 128)
                pltpu.sync_copy(
                    table_t.at[pl.ds(r0, GRP), pl.ds(c0, WIN)], buf)
                for s in range(GRP):
                    pltpu.sync_copy(
                        buf.at[s],
                        flat_out.at[pl.ds((r0 + s) * STRIDE + c0, WIN)])

            @pl.when(win == NFULL)
            def _last():
                c0 = pl.multiple_of(NFULL * WIN, 128)
                pltpu.sync_copy(
                    table_t.at[pl.ds(r0, GRP), pl.ds(c0, WIN2)], buf2)
                for s in range(GRP):
                    pltpu.sync_copy(
                        buf2.at[s],
                        flat_out.at[pl.ds((r0 + s) * STRIDE + c0, WIN2)])

        @pl.when(item == NITEM - 1)
        def _tail():
            pltpu.sync_copy(tail_t, tail_buf)
            for s in range(EMBED_DIM):
                pltpu.sync_copy(
                    tail_buf.at[s],
                    flat_out.at[pl.ds(s * STRIDE + COVER, TAIL)])

        return carry

    lax.fori_loop(0, ITER, round_, 0)


def _gather_body(flat_tab, bias_tab, nid, nnid,
                 score_out, embu_t_out, embv_t_out, bias_out,
                 idx_u, idx_v, idx_d, cols_u, cols_v, bias_vm, score_vm,
                 sem_u, sem_v, sem_b, sem_o):
    wid = lax.axis_index("s") * NUM_CORES + lax.axis_index("c")
    base = wid * B_PER_W

    pltpu.sync_copy(nid.at[pl.ds(base, B_PER_W)], idx_u)
    pltpu.sync_copy(nnid.at[pl.ds(base, B_PER_W)], idx_v)
    cp_b = pltpu.async_copy(bias_tab.at[idx_v], bias_vm, sem_b)
    cps = []
    for d in range(EMBED_DIM):
        plane = flat_tab.at[pl.ds(d * STRIDE, N_NODE)]
        cps.append(pltpu.async_copy(plane.at[idx_u], cols_u.at[d], sem_u))
        cps.append(pltpu.async_copy(plane.at[idx_v], cols_v.at[d], sem_v))
    for cp in cps:
        cp.wait()
    cp_b.wait()

    ocs = [pltpu.async_copy(bias_vm, bias_out.at[pl.ds(base, B_PER_W)], sem_o)]
    for d in range(EMBED_DIM):
        ocs.append(pltpu.async_copy(
            cols_u.at[d], embu_t_out.at[d, pl.ds(base, B_PER_W)], sem_o))
        ocs.append(pltpu.async_copy(
            cols_v.at[d], embv_t_out.at[d, pl.ds(base, B_PER_W)], sem_o))

    def block(j, carry):
        sl = pl.ds(j * NUM_LANES, NUM_LANES)
        acc = bias_vm[sl]
        for d in range(EMBED_DIM):
            acc = acc + cols_u[d, sl] * cols_v[d, sl]
        score_vm[sl] = jnp.clip(acc, -10.0, 10.0)
        return carry

    lax.fori_loop(0, NBLK, block, 0)

    pltpu.sync_copy(score_vm, score_out.at[pl.ds(base, B_PER_W)])
    for oc in ocs:
        oc.wait()


@jax.jit
def kernel(embedding_matrix, bias_vector, node_id, node_neighbor_id):
    mesh = plsc.VectorSubcoreMesh(core_axis_name="c", subcore_axis_name="s")

    table_t = embedding_matrix.T
    detile = functools.partial(
        pl.kernel,
        mesh=mesh,
        out_type=jax.ShapeDtypeStruct((EMBED_DIM * STRIDE,), jnp.float32),
        scratch_types=[
            pltpu.VMEM((GRP, WIN), jnp.float32),
            pltpu.VMEM((GRP, WIN2), jnp.float32),
            pltpu.VMEM((EMBED_DIM, TAIL), jnp.float32),
        ],
    )(_detile_body)
    flat_tab = detile(table_t, lax.slice(table_t, (0, COVER),
                                         (EMBED_DIM, N_NODE)))

    gather = functools.partial(
        pl.kernel,
        mesh=mesh,
        compiler_params=pltpu.CompilerParams(use_tc_tiling_on_sc=False),
        out_type=[
            jax.ShapeDtypeStruct((BATCH,), jnp.float32),
            jax.ShapeDtypeStruct((EMBED_DIM, BATCH), jnp.float32),
            jax.ShapeDtypeStruct((EMBED_DIM, BATCH), jnp.float32),
            jax.ShapeDtypeStruct((BATCH,), jnp.float32),
        ],
        scratch_types=[
            pltpu.VMEM((B_PER_W,), jnp.int32),
            pltpu.VMEM((B_PER_W,), jnp.int32),
            pltpu.VMEM((B_PER_W,), jnp.int32),
            pltpu.VMEM((EMBED_DIM, B_PER_W), jnp.float32),
            pltpu.VMEM((EMBED_DIM, B_PER_W), jnp.float32),
            pltpu.VMEM((B_PER_W,), jnp.float32),
            pltpu.VMEM((B_PER_W,), jnp.float32),
            pltpu.SemaphoreType.DMA,
            pltpu.SemaphoreType.DMA,
            pltpu.SemaphoreType.DMA,
            pltpu.SemaphoreType.DMA,
        ],
    )(_gather_body)
    score, embu_t, embv_t, bias = gather(
        flat_tab,
        bias_vector,
        node_id.astype(jnp.int32),
        node_neighbor_id.astype(jnp.int32),
    )
    return (score, embu_t.T, embv_t.T, bias)
